# baseline (device time: 25171 ns/iter reference)
import os

import jax
import jax.numpy as jnp
from jax import lax
from jax.experimental import pallas as pl
from jax.experimental.pallas import tpu as pltpu

_NO_COMM = os.environ.get("KNC") == "1"
_NO_COMPUTE = os.environ.get("KNP") == "1"

N_DEV = 4
B, SQ, SKV, HQ, DH = 2, 256, 256, 16, 64
H_PER = HQ // N_DEV
DM = 512
NBF = 2
CH = SQ // NBF


def kernel(x, Wq, K_ext, V_ext, Wo):
    my_i = lax.axis_index("i")
    h0 = my_i * H_PER
    K_sh = lax.dynamic_slice_in_dim(K_ext, h0, H_PER, axis=2)
    V_sh = lax.dynamic_slice_in_dim(V_ext, h0, H_PER, axis=2)
    K_sh = K_sh.reshape(B, SKV, H_PER * DH)
    V_sh = V_sh.reshape(B, SKV, H_PER * DH)

    def body(x_ref, wq_ref, k_ref, v_ref, wo_ref, out_ref,
             comm_ref, send_sems, recv_sems):
        my_pos = lax.axis_index("i")
        pn = jnp.bitwise_xor(my_pos, 1)
        pd = jnp.bitwise_xor(my_pos, 2)

        barrier_sem = pltpu.get_barrier_semaphore()
        for p in (pn, pd):
            pl.semaphore_signal(barrier_sem, inc=1, device_id=(p,),
                                device_id_type=pl.DeviceIdType.MESH)

        xf = x_ref[...].astype(jnp.bfloat16).reshape(B * SQ, DM)
        q = jnp.dot(xf, wq_ref[...].astype(jnp.bfloat16),
                    preferred_element_type=jnp.float32)
        q = q.astype(jnp.bfloat16).reshape(B, SQ, H_PER, DH)
        wob = wo_ref[...].astype(jnp.bfloat16)

        qi = lax.broadcasted_iota(jnp.int32, (SQ, SKV), 0)
        ki = lax.broadcasted_iota(jnp.int32, (SQ, SKV), 1)
        mask = (jnp.abs(qi - ki) <= 128) | (ki < 32) | (qi < 32)

        PART0 = (pn, pd)
        PART1 = (pd, pn)

        def mk(stage, b, bf):
            sl = pl.ds(bf * CH, CH)
            return pltpu.make_async_remote_copy(
                src_ref=comm_ref.at[2 * stage, b, sl],
                dst_ref=comm_ref.at[2 * stage + 1, b, sl],
                send_sem=send_sems.at[stage, b, bf],
                recv_sem=recv_sems.at[stage, b, bf],
                device_id=((PART0, PART1)[stage][bf],),
                device_id_type=pl.DeviceIdType.MESH,
            )

        rdma0 = [[mk(0, b, bf) for bf in range(NBF)] for b in range(B)]
        rdma1 = [[mk(1, b, bf) for bf in range(NBF)] for b in range(B)]

        CHUNKS = [(b, bf) for b in range(B) for bf in range(NBF)]
        LAG = 2

        def stage1_process(b, bf):
            sl = pl.ds(bf * CH, CH)
            rdma0[b][bf].wait_recv()
            comm_ref[2, b, sl] = comm_ref[0, b, sl] + comm_ref[1, b, sl]
            rdma1[b][bf].start()

        barrier_waited = False
        kv = {}
        for i, (b, bf) in enumerate(CHUNKS):
            if b not in kv:
                kv[b] = (k_ref[b].astype(jnp.bfloat16).reshape(SKV, H_PER, DH),
                         v_ref[b].astype(jnp.bfloat16).reshape(SKV, H_PER, DH))
            kb, vb = kv[b]
            sl = pl.ds(bf * CH, CH)
            if _NO_COMPUTE:
                part = x_ref[b, sl].astype(jnp.bfloat16)
            else:
                mrows = mask[bf * CH:(bf + 1) * CH]
                ctx_heads = []
                for h in range(H_PER):
                    s = lax.dot_general(
                        q[b, bf * CH:(bf + 1) * CH, h, :], kb[:, h, :],
                        (((1,), (1,)), ((), ())),
                        preferred_element_type=jnp.float32,
                    ) * 0.125
                    w = jnp.where(mrows, jnp.exp(s), 0.0)
                    recip = 1.0 / w.sum(axis=-1, keepdims=True)
                    ctx_heads.append(
                        jnp.dot(w.astype(jnp.bfloat16), vb[:, h, :],
                                preferred_element_type=jnp.float32)
                        * recip)
                ctx = jnp.concatenate(ctx_heads, axis=1).astype(jnp.bfloat16)
                part = jnp.dot(ctx, wob,
                               preferred_element_type=jnp.float32
                               ).astype(jnp.bfloat16)
            comm_ref[0, b, sl] = part
            if not barrier_waited:
                pl.semaphore_wait(barrier_sem, 2)
                barrier_waited = True
            if not _NO_COMM:
                rdma0[b][bf].start()
                if i >= LAG:
                    stage1_process(*CHUNKS[i - LAG])

        if _NO_COMM:
            for b in range(B):
                out_ref[b] = comm_ref[0, b]
            return

        for i in range(len(CHUNKS) - LAG, len(CHUNKS)):
            stage1_process(*CHUNKS[i])

        for b, bf in CHUNKS:
            sl = pl.ds(bf * CH, CH)
            rdma1[b][bf].wait_recv()
            out_ref[b, sl] = comm_ref[2, b, sl] + comm_ref[3, b, sl]

        for b, bf in CHUNKS:
            rdma0[b][bf].wait_send()
            rdma1[b][bf].wait_send()

    out_shape = jax.ShapeDtypeStruct((B, SQ, DM), jnp.bfloat16)
    return pl.pallas_call(
        body,
        out_shape=out_shape,
        in_specs=[pl.BlockSpec(memory_space=pltpu.VMEM)] * 5,
        out_specs=pl.BlockSpec(memory_space=pltpu.VMEM),
        scratch_shapes=[
            pltpu.VMEM((4, B, SQ, DM), jnp.bfloat16),
            pltpu.SemaphoreType.DMA((2, B, NBF)),
            pltpu.SemaphoreType.DMA((2, B, NBF)),
        ],
        compiler_params=pltpu.CompilerParams(collective_id=0),
    )(x, Wq, K_sh, V_sh, Wo)


# device time: 20354 ns/iter; 1.2367x vs baseline; 1.2367x over previous
import os

import jax
import jax.numpy as jnp
from jax import lax
from jax.experimental import pallas as pl
from jax.experimental.pallas import tpu as pltpu

_NO_COMM = os.environ.get("KNC") == "1"
_NO_COMPUTE = os.environ.get("KNP") == "1"

N_DEV = 4
B, SQ, SKV, HQ, DH = 2, 256, 256, 16, 64
H_PER = HQ // N_DEV
DM = 512
NBF = 2
CH = SQ // NBF


def kernel(x, Wq, K_ext, V_ext, Wo):
    my_i = lax.axis_index("i")
    h0 = my_i * H_PER
    K_sh = lax.dynamic_slice_in_dim(K_ext, h0, H_PER, axis=2).astype(jnp.bfloat16)
    V_sh = lax.dynamic_slice_in_dim(V_ext, h0, H_PER, axis=2).astype(jnp.bfloat16)

    def body(x_ref, wq_ref, k_ref, v_ref, wo_ref, out_ref,
             comm_ref, send_sems, recv_sems):
        my_pos = lax.axis_index("i")
        pn = jnp.bitwise_xor(my_pos, 1)
        pd = jnp.bitwise_xor(my_pos, 2)

        barrier_sem = pltpu.get_barrier_semaphore()
        for p in (pn, pd):
            pl.semaphore_signal(barrier_sem, inc=1, device_id=(p,),
                                device_id_type=pl.DeviceIdType.MESH)

        xf = x_ref[...].astype(jnp.bfloat16).reshape(B * SQ, DM)
        q = jnp.dot(xf, wq_ref[...].astype(jnp.bfloat16),
                    preferred_element_type=jnp.float32)
        q = q.astype(jnp.bfloat16).reshape(B, SQ, H_PER, DH)
        wob = wo_ref[...].astype(jnp.bfloat16)

        qi = lax.broadcasted_iota(jnp.int32, (SQ, SKV), 0)
        ki = lax.broadcasted_iota(jnp.int32, (SQ, SKV), 1)
        mask = (jnp.abs(qi - ki) <= 128) | (ki < 32) | (qi < 32)

        PART0 = tuple(pn if bf % 2 == 0 else pd for bf in range(NBF))
        PART1 = tuple(pd if bf % 2 == 0 else pn for bf in range(NBF))

        def mk(stage, b, bf):
            sl = pl.ds(bf * CH, CH)
            return pltpu.make_async_remote_copy(
                src_ref=comm_ref.at[2 * stage, b, sl],
                dst_ref=comm_ref.at[2 * stage + 1, b, sl],
                send_sem=send_sems.at[stage, b, bf],
                recv_sem=recv_sems.at[stage, b, bf],
                device_id=((PART0, PART1)[stage][bf],),
                device_id_type=pl.DeviceIdType.MESH,
            )

        rdma0 = [[mk(0, b, bf) for bf in range(NBF)] for b in range(B)]
        rdma1 = [[mk(1, b, bf) for bf in range(NBF)] for b in range(B)]

        CHUNKS = [(b, bf) for b in range(B) for bf in range(NBF)]
        LAG = len(CHUNKS)

        def stage1_process(b, bf):
            sl = pl.ds(bf * CH, CH)
            rdma0[b][bf].wait_recv()
            comm_ref[2, b, sl] = comm_ref[0, b, sl] + comm_ref[1, b, sl]
            rdma1[b][bf].start()

        barrier_waited = False
        kv = {}
        for i, (b, bf) in enumerate(CHUNKS):
            if b not in kv:
                kv[b] = (k_ref[b], v_ref[b])
            kb, vb = kv[b]
            sl = pl.ds(bf * CH, CH)
            if _NO_COMPUTE:
                part = x_ref[b, sl].astype(jnp.bfloat16)
            else:
                mrows = mask[bf * CH:(bf + 1) * CH]
                ctx_heads = []
                for h in range(H_PER):
                    s = lax.dot_general(
                        q[b, bf * CH:(bf + 1) * CH, h, :], kb[:, h, :],
                        (((1,), (1,)), ((), ())),
                        preferred_element_type=jnp.float32,
                    ) * 0.125
                    w = jnp.where(mrows, jnp.exp(s), 0.0)
                    recip = 1.0 / w.sum(axis=-1, keepdims=True)
                    ctx_heads.append(
                        jnp.dot(w.astype(jnp.bfloat16), vb[:, h, :],
                                preferred_element_type=jnp.float32)
                        * recip)
                ctx = jnp.concatenate(ctx_heads, axis=1).astype(jnp.bfloat16)
                part = jnp.dot(ctx, wob,
                               preferred_element_type=jnp.float32
                               ).astype(jnp.bfloat16)
            comm_ref[0, b, sl] = part
            if not barrier_waited:
                pl.semaphore_wait(barrier_sem, 2)
                barrier_waited = True
            if not _NO_COMM:
                rdma0[b][bf].start()
                if i >= LAG:
                    stage1_process(*CHUNKS[i - LAG])

        if _NO_COMM:
            for b in range(B):
                out_ref[b] = comm_ref[0, b]
            return

        for i in range(len(CHUNKS) - LAG, len(CHUNKS)):
            stage1_process(*CHUNKS[i])

        for b, bf in CHUNKS:
            sl = pl.ds(bf * CH, CH)
            rdma1[b][bf].wait_recv()
            out_ref[b, sl] = comm_ref[2, b, sl] + comm_ref[3, b, sl]

        for b, bf in CHUNKS:
            rdma0[b][bf].wait_send()
            rdma1[b][bf].wait_send()

    out_shape = jax.ShapeDtypeStruct((B, SQ, DM), jnp.bfloat16)
    return pl.pallas_call(
        body,
        out_shape=out_shape,
        in_specs=[pl.BlockSpec(memory_space=pltpu.VMEM)] * 5,
        out_specs=pl.BlockSpec(memory_space=pltpu.VMEM),
        scratch_shapes=[
            pltpu.VMEM((4, B, SQ, DM), jnp.bfloat16),
            pltpu.SemaphoreType.DMA((2, B, NBF)),
            pltpu.SemaphoreType.DMA((2, B, NBF)),
        ],
        compiler_params=pltpu.CompilerParams(collective_id=0),
    )(x, Wq, K_sh, V_sh, Wo)
